# baseline (speedup 1.0000x reference)
import functools

import jax
import jax.numpy as jnp
from jax import lax
from jax.experimental import pallas as pl
from jax.experimental.pallas import tpu as pltpu
from jax.experimental.pallas import tpu_sc as plsc

_N = 10000
_E = 320000
_D = 128
_H = 4
_ED = 16
_HD = _H * _D

_NC = 2
_NS = 16
_NW = _NC * _NS
_EPW = _E // _NW
_C1 = 16
_NCHUNK1 = _EPW // _C1
_C = 80
_NCHUNK = _EPW // _C
_QGW = 640
_RW = 128
_NSLAB = 10240
_RPS = _NSLAB // _NS
_ZR = 128

_BN = 400
_NB = _N // _BN


def _tca_body(x_ref, wq_ref, bq_ref, wk_ref, bk_ref, wv_ref, bv_ref,
              wet_ref, qg_ref, k4_ref, v4_ref):
    xb = x_ref[...]
    scale = float(_D) ** -0.5
    q = (jnp.dot(xb, wq_ref[...], preferred_element_type=jnp.float32)
         + bq_ref[...]) * scale
    k = jnp.dot(xb, wk_ref[...], preferred_element_type=jnp.float32) + bk_ref[...]
    v = jnp.dot(xb, wv_ref[...], preferred_element_type=jnp.float32) + bv_ref[...]
    g = jnp.dot(q, wet_ref[...], preferred_element_type=jnp.float32)
    qg_ref[:, :_HD] = q
    qg_ref[:, _HD:_HD + _H * _ED] = g
    qg_ref[:, _HD + _H * _ED:] = jnp.zeros((_BN, _QGW - _HD - _H * _ED),
                                           jnp.float32)
    k4_ref[...] = k
    v4_ref[...] = v


def _tca(x, wq, bq, wk, bk, wv, bv, wet):
    w_spec = lambda shp: pl.BlockSpec(shp, lambda i: (0, 0))
    return pl.pallas_call(
        _tca_body,
        grid=(_NB,),
        in_specs=[
            pl.BlockSpec((_BN, _D), lambda i: (i, 0)),
            w_spec((_D, _HD)), w_spec((1, _HD)),
            w_spec((_D, _HD)), w_spec((1, _HD)),
            w_spec((_D, _HD)), w_spec((1, _HD)),
            w_spec((_HD, _H * _ED)),
        ],
        out_specs=[
            pl.BlockSpec((_BN, _QGW), lambda i: (i, 0)),
            pl.BlockSpec((_BN, _HD), lambda i: (i, 0)),
            pl.BlockSpec((_BN, _HD), lambda i: (i, 0)),
        ],
        out_shape=[
            jax.ShapeDtypeStruct((_N, _QGW), jnp.float32),
            jax.ShapeDtypeStruct((_N, _HD), jnp.float32),
            jax.ShapeDtypeStruct((_N, _HD), jnp.float32),
        ],
    )(x, wq, bq, wk, bk, wv, bv, wet)


def _tcb_body(zs_ref, oh_ref, x_ref, we_ref, wskip_ref, bskip_ref, wproj_ref,
              bproj_ref, gamma_ref, beta_ref, o_ref):
    xb = x_ref[...]
    we = we_ref[...]
    zs = zs_ref[0] + zs_ref[1]
    outs = []
    for h in range(_H):
        o = oh_ref[h, 0] + oh_ref[h, 1]
        z = zs[:, h * _ED:(h + 1) * _ED]
        s = zs[:, _H * _ED + h * _ED:_H * _ED + h * _ED + 1]
        num = o + jnp.dot(z, we[:, h * _D:(h + 1) * _D],
                          preferred_element_type=jnp.float32)
        outs.append(num / (s + 1e-16))
    out = jnp.concatenate(outs, axis=1)
    out = out + jnp.dot(xb, wskip_ref[...],
                        preferred_element_type=jnp.float32) + bskip_ref[...]
    hh = jnp.dot(out, wproj_ref[...],
                 preferred_element_type=jnp.float32) + bproj_ref[...]
    hh = jnp.where(hh > 0, hh, jnp.exp(jnp.minimum(hh, 0.0)) - 1.0)
    y = xb + hh
    mu = jnp.mean(y, axis=1, keepdims=True)
    var = jnp.mean((y - mu) * (y - mu), axis=1, keepdims=True)
    y = (y - mu) * lax.rsqrt(var + 1e-5)
    o_ref[...] = y * gamma_ref[...] + beta_ref[...]


def _tcb(zs, oh, x, we, wskip, bskip, wproj, bproj, gamma, beta):
    return pl.pallas_call(
        _tcb_body,
        grid=(_NB,),
        in_specs=[
            pl.BlockSpec((_NC, _BN, _RW), lambda i: (0, i, 0)),
            pl.BlockSpec((_H, _NC, _BN, _RW), lambda i: (0, 0, i, 0)),
            pl.BlockSpec((_BN, _D), lambda i: (i, 0)),
            pl.BlockSpec((_ED, _HD), lambda i: (0, 0)),
            pl.BlockSpec((_D, _HD), lambda i: (0, 0)),
            pl.BlockSpec((1, _HD), lambda i: (0, 0)),
            pl.BlockSpec((_HD, _D), lambda i: (0, 0)),
            pl.BlockSpec((1, _D), lambda i: (0, 0)),
            pl.BlockSpec((1, _D), lambda i: (0, 0)),
            pl.BlockSpec((1, _D), lambda i: (0, 0)),
        ],
        out_specs=pl.BlockSpec((_BN, _D), lambda i: (i, 0)),
        out_shape=jax.ShapeDtypeStruct((_N, _D), jnp.float32),
    )(zs, oh, x, we, wskip, bskip, wproj, bproj, gamma, beta)


_SR = 64
_SR1 = 16


def _fill_zeros(stg, rows):
    def _zb(i, carry):
        for t in range(_RW // 16):
            stg[i, pl.ds(t * 16, 16)] = jnp.zeros((16,), jnp.float32)
        return carry
    lax.fori_loop(0, rows, _zb, 0)


def _zero_slab(slab, stg, sid, rows):
    for j in range(_RPS // rows):
        pltpu.sync_copy(stg, slab.at[pl.ds(sid * _RPS + j * rows, rows)])


def _slab_to_hbm(slab, stg, out_hbm_rows, sid, rows):
    for j in range(_RPS // rows):
        r = sid * _RPS + j * rows
        pltpu.sync_copy(slab.at[pl.ds(r, rows)], stg)
        pltpu.sync_copy(stg, out_hbm_rows.at[pl.ds(r, rows)])


def _sc_p1_body(qg_hbm, k4_hbm, ea_hbm, src_hbm, dst_hbm,
                zs_hbm, p_hbm,
                slab, stg, dstbuf, srcbuf, qgbuf, k4buf, eabuf, resbuf,
                pbuf, sem_a, sem_b):
    cid = lax.axis_index("c")
    sid = lax.axis_index("s")
    wid = cid * _NS + sid
    lane = lax.iota(jnp.int32, 16)
    rot = [(lane + sh) & 15 for sh in (8, 4, 2, 1)]

    _fill_zeros(stg, _SR1)
    _zero_slab(slab, stg, sid, _SR1)
    plsc.subcore_barrier()

    def _base(cidx):
        return pl.multiple_of(wid * _EPW + cidx * _C1, 8)

    def _issue_idx(cidx, s):
        b = _base(cidx)
        return (
            pltpu.async_copy(dst_hbm.at[pl.ds(b, _C1)], dstbuf.at[s], sem_a),
            pltpu.async_copy(src_hbm.at[pl.ds(b, _C1)], srcbuf.at[s], sem_a),
            pltpu.async_copy(ea_hbm.at[pl.ds(b * _ED, _C1 * _ED)],
                             eabuf.at[s], sem_a),
        )

    def _wait_idx(s):
        pltpu.make_async_copy(dst_hbm.at[pl.ds(0, _C1)],
                              dstbuf.at[s], sem_a).wait()
        pltpu.make_async_copy(src_hbm.at[pl.ds(0, _C1)],
                              srcbuf.at[s], sem_a).wait()
        pltpu.make_async_copy(ea_hbm.at[pl.ds(0, _C1 * _ED)],
                              eabuf.at[s], sem_a).wait()

    def _issue_gather(s):
        pltpu.async_copy(qg_hbm.at[dstbuf.at[s]], qgbuf.at[s], sem_b)
        pltpu.async_copy(k4_hbm.at[srcbuf.at[s]], k4buf.at[s], sem_b)

    def _wait_gather(s):
        pltpu.make_async_copy(qg_hbm.at[dstbuf.at[s]],
                              qgbuf.at[s], sem_b).wait()
        pltpu.make_async_copy(k4_hbm.at[srcbuf.at[s]],
                              k4buf.at[s], sem_b).wait()

    for hnd in _issue_idx(0, 0):
        hnd.wait()
    _issue_gather(0)

    def _chunk(cidx, carry):
        s = lax.rem(cidx, 2)
        o = 1 - s
        not_last = cidx < _NCHUNK1 - 1

        @pl.when(not_last)
        def _():
            _issue_idx(cidx + 1, o)

        _wait_gather(s)

        def _edge(e, ecarry):
            ea = eabuf[s, pl.ds(e * _ED, _ED)]
            pv = []
            for h in range(_H):
                acc = ea * qgbuf[s, e, pl.ds(_HD + h * _ED, 16)]
                for r in range(8):
                    acc = acc + (qgbuf[s, e, pl.ds(h * _D + r * 16, 16)]
                                 * k4buf[s, e, pl.ds(h * _D + r * 16, 16)])
                for rv in rot:
                    acc = acc + jnp.take(acc, rv)
                p = jnp.exp(acc)
                resbuf[e, pl.ds(h * _ED, 16)] = p * ea
                resbuf[e, pl.ds(_H * _ED + h * _ED, 16)] = p
                pv.append(p)
            ps = pv[0]
            for h in range(1, _H):
                ps = jnp.where(lane == h, pv[h], ps)
            pbuf[pl.ds(e * 16, 16)] = ps
            return ecarry
        lax.fori_loop(0, _C1, _edge, 0)

        @pl.when(not_last)
        def _():
            _wait_idx(o)
            _issue_gather(o)

        pltpu.sync_copy(resbuf, slab.at[dstbuf.at[s]], add=True)
        pltpu.sync_copy(pbuf, p_hbm.at[pl.ds(_base(cidx) * 16, _C1 * 16)])
        return carry
    lax.fori_loop(0, _NCHUNK1, _chunk, 0)
    plsc.subcore_barrier()

    _slab_to_hbm(slab, stg, zs_hbm.at[cid], sid, _SR1)


_sc_p1 = functools.partial(
    pl.kernel,
    out_type=[
        jax.ShapeDtypeStruct((_NC, _NSLAB, _RW), jnp.float32),
        jax.ShapeDtypeStruct((_E * 16,), jnp.float32),
    ],
    mesh=plsc.VectorSubcoreMesh(core_axis_name="c", subcore_axis_name="s"),
    scratch_types=[
        pltpu.VMEM_SHARED((_NSLAB, _RW), jnp.float32),
        pltpu.VMEM((_SR1, _RW), jnp.float32),
        pltpu.VMEM((2, _C1), jnp.int32),
        pltpu.VMEM((2, _C1), jnp.int32),
        pltpu.VMEM((2, _C1, _QGW), jnp.float32),
        pltpu.VMEM((2, _C1, _HD), jnp.float32),
        pltpu.VMEM((2, _C1 * _ED), jnp.float32),
        pltpu.VMEM((_C1, _RW), jnp.float32),
        pltpu.VMEM((_C1 * 16,), jnp.float32),
        pltpu.SemaphoreType.DMA,
        pltpu.SemaphoreType.DMA,
    ],
)(_sc_p1_body)


def _sc_p2_body(v4_hbm, p_hbm, src_hbm, dst_hbm, oh_hbm,
                slab, stg, dstbuf, srcbuf, vidx, vbuf, pbuf, resbuf,
                sem_a, sem_b):
    cid = lax.axis_index("c")
    sid = lax.axis_index("s")
    wid = cid * _NS + sid

    def _base(cidx):
        return pl.multiple_of(wid * _EPW + cidx * _C, 8)

    def _issue_idx(cidx, s):
        b = _base(cidx)
        return (
            pltpu.async_copy(dst_hbm.at[pl.ds(b, _C)], dstbuf.at[s], sem_a),
            pltpu.async_copy(src_hbm.at[pl.ds(b, _C)], srcbuf.at[s], sem_a),
            pltpu.async_copy(p_hbm.at[pl.ds(b * 16, _C * 16)],
                             pbuf.at[s], sem_a),
        )

    def _wait_idx(s):
        pltpu.make_async_copy(dst_hbm.at[pl.ds(0, _C)],
                              dstbuf.at[s], sem_a).wait()
        pltpu.make_async_copy(src_hbm.at[pl.ds(0, _C)],
                              srcbuf.at[s], sem_a).wait()
        pltpu.make_async_copy(p_hbm.at[pl.ds(0, _C * 16)],
                              pbuf.at[s], sem_a).wait()

    for h in range(_H):
        _fill_zeros(stg, _SR)
        _zero_slab(slab, stg, sid, _SR)
        plsc.subcore_barrier()

        def _vidx_and_gather(s):
            for t in range(_C // 16):
                sv = srcbuf[s, pl.ds(t * 16, 16)]
                vidx[s, pl.ds(t * 16, 16)] = sv * _H + h
            pltpu.async_copy(v4_hbm.at[vidx.at[s]], vbuf.at[s], sem_b)

        def _wait_gather(s):
            pltpu.make_async_copy(v4_hbm.at[vidx.at[s]],
                                  vbuf.at[s], sem_b).wait()

        for hnd in _issue_idx(0, 0):
            hnd.wait()
        _vidx_and_gather(0)

        hsel = jnp.full((16,), h, jnp.int32)

        def _chunk(cidx, carry):
            s = lax.rem(cidx, 2)
            o = 1 - s
            not_last = cidx < _NCHUNK - 1

            @pl.when(not_last)
            def _():
                _issue_idx(cidx + 1, o)

            _wait_gather(s)

            def _edge(e, ecarry):
                prow = pbuf[s, pl.ds(e * 16, 16)]
                p = jnp.take(prow, hsel)
                for r in range(8):
                    resbuf[e, pl.ds(r * 16, 16)] = (
                        p * vbuf[s, e, pl.ds(r * 16, 16)])
                return ecarry
            lax.fori_loop(0, _C, _edge, 0)

            @pl.when(not_last)
            def _():
                _wait_idx(o)
                _vidx_and_gather(o)

            pltpu.sync_copy(resbuf, slab.at[dstbuf.at[s]], add=True)
            return carry
        lax.fori_loop(0, _NCHUNK, _chunk, 0)
        plsc.subcore_barrier()

        _slab_to_hbm(slab, stg, oh_hbm.at[h, cid], sid, _SR)


_sc_p2 = functools.partial(
    pl.kernel,
    out_type=jax.ShapeDtypeStruct((_H, _NC, _NSLAB, _RW), jnp.float32),
    mesh=plsc.VectorSubcoreMesh(core_axis_name="c", subcore_axis_name="s"),
    scratch_types=[
        pltpu.VMEM_SHARED((_NSLAB, _RW), jnp.float32),
        pltpu.VMEM((_SR, _RW), jnp.float32),
        pltpu.VMEM((2, _C), jnp.int32),
        pltpu.VMEM((2, _C), jnp.int32),
        pltpu.VMEM((2, _C), jnp.int32),
        pltpu.VMEM((2, _C, _D), jnp.float32),
        pltpu.VMEM((2, _C * 16), jnp.float32),
        pltpu.VMEM((_C, _RW), jnp.float32),
        pltpu.SemaphoreType.DMA,
        pltpu.SemaphoreType.DMA,
    ],
)(_sc_p2_body)


def kernel(x, edge_index, edge_attr, Wq, bq, Wk, bk, Wv, bv, We, Wskip,
           bskip, Wproj, bproj, gamma, beta):
    src = edge_index[0].astype(jnp.int32)
    dst = edge_index[1].astype(jnp.int32)
    ea1d = edge_attr.reshape(-1)
    for l in range(3):
        we_l = We[l]
        wr = we_l.reshape(_ED, _H, _D)
        wet = jnp.zeros((_HD, _H * _ED), jnp.float32)
        for h in range(_H):
            wet = wet.at[h * _D:(h + 1) * _D,
                         h * _ED:(h + 1) * _ED].set(wr[:, h, :].T)
        qg, k4, v4 = _tca(x, Wq[l], bq[l].reshape(1, -1), Wk[l],
                          bk[l].reshape(1, -1), Wv[l], bv[l].reshape(1, -1),
                          wet)
        zs, p1d = _sc_p1(qg, k4, ea1d, src, dst)
        oh = _sc_p2(v4.reshape(_N * _H, _D), p1d, src, dst)
        x = _tcb(zs, oh, x, we_l, Wskip[l], bskip[l].reshape(1, -1),
                 Wproj[l], bproj[l].reshape(1, -1), gamma[l].reshape(1, -1),
                 beta[l].reshape(1, -1))
    return x


# speedup vs baseline: 1.1623x; 1.1623x over previous
import functools

import jax
import jax.numpy as jnp
from jax import lax
from jax.experimental import pallas as pl
from jax.experimental.pallas import tpu as pltpu
from jax.experimental.pallas import tpu_sc as plsc

_N = 10000
_E = 320000
_D = 128
_H = 4
_ED = 16
_HD = _H * _D

_NC = 2
_NS = 16
_NW = _NC * _NS
_EPW = _E // _NW
_C1 = 16
_NCHUNK1 = _EPW // _C1
_C = 80
_NCHUNK = _EPW // _C
_QGW = 640
_RW = 128
_NSLAB = 10240
_RPS = _NSLAB // _NS
_ZR = 128

_BN = 400
_NB = _N // _BN


def _tca_body(x_ref, wq_ref, bq_ref, wk_ref, bk_ref, wv_ref, bv_ref,
              wet_ref, qg_ref, k4_ref, v4_ref):
    xb = x_ref[...]
    scale = float(_D) ** -0.5
    q = (jnp.dot(xb, wq_ref[...], preferred_element_type=jnp.float32)
         + bq_ref[...]) * scale
    k = jnp.dot(xb, wk_ref[...], preferred_element_type=jnp.float32) + bk_ref[...]
    v = jnp.dot(xb, wv_ref[...], preferred_element_type=jnp.float32) + bv_ref[...]
    g = jnp.dot(q, wet_ref[...], preferred_element_type=jnp.float32)
    qg_ref[:, :_HD] = q
    qg_ref[:, _HD:_HD + _H * _ED] = g
    qg_ref[:, _HD + _H * _ED:] = jnp.zeros((_BN, _QGW - _HD - _H * _ED),
                                           jnp.float32)
    k4_ref[...] = k
    v4_ref[...] = v


def _tca(x, wq, bq, wk, bk, wv, bv, wet):
    w_spec = lambda shp: pl.BlockSpec(shp, lambda i: (0, 0))
    return pl.pallas_call(
        _tca_body,
        grid=(_NB,),
        in_specs=[
            pl.BlockSpec((_BN, _D), lambda i: (i, 0)),
            w_spec((_D, _HD)), w_spec((1, _HD)),
            w_spec((_D, _HD)), w_spec((1, _HD)),
            w_spec((_D, _HD)), w_spec((1, _HD)),
            w_spec((_HD, _H * _ED)),
        ],
        out_specs=[
            pl.BlockSpec((_BN, _QGW), lambda i: (i, 0)),
            pl.BlockSpec((_BN, _HD), lambda i: (i, 0)),
            pl.BlockSpec((_BN, _HD), lambda i: (i, 0)),
        ],
        out_shape=[
            jax.ShapeDtypeStruct((_N, _QGW), jnp.float32),
            jax.ShapeDtypeStruct((_N, _HD), jnp.float32),
            jax.ShapeDtypeStruct((_N, _HD), jnp.float32),
        ],
    )(x, wq, bq, wk, bk, wv, bv, wet)


def _tcb_body(zs_ref, oh_ref, x_ref, we_ref, wskip_ref, bskip_ref, wproj_ref,
              bproj_ref, gamma_ref, beta_ref, o_ref):
    xb = x_ref[...]
    we = we_ref[...]
    zs = zs_ref[0] + zs_ref[1]
    outs = []
    for h in range(_H):
        o = oh_ref[h, 0] + oh_ref[h, 1]
        z = zs[:, h * _ED:(h + 1) * _ED]
        s = zs[:, _H * _ED + h * _ED:_H * _ED + h * _ED + 1]
        num = o + jnp.dot(z, we[:, h * _D:(h + 1) * _D],
                          preferred_element_type=jnp.float32)
        outs.append(num / (s + 1e-16))
    out = jnp.concatenate(outs, axis=1)
    out = out + jnp.dot(xb, wskip_ref[...],
                        preferred_element_type=jnp.float32) + bskip_ref[...]
    hh = jnp.dot(out, wproj_ref[...],
                 preferred_element_type=jnp.float32) + bproj_ref[...]
    hh = jnp.where(hh > 0, hh, jnp.exp(jnp.minimum(hh, 0.0)) - 1.0)
    y = xb + hh
    mu = jnp.mean(y, axis=1, keepdims=True)
    var = jnp.mean((y - mu) * (y - mu), axis=1, keepdims=True)
    y = (y - mu) * lax.rsqrt(var + 1e-5)
    o_ref[...] = y * gamma_ref[...] + beta_ref[...]


def _tcb(zs, oh, x, we, wskip, bskip, wproj, bproj, gamma, beta):
    return pl.pallas_call(
        _tcb_body,
        grid=(_NB,),
        in_specs=[
            pl.BlockSpec((_NC, _BN, _RW), lambda i: (0, i, 0)),
            pl.BlockSpec((_H, _NC, _BN, _RW), lambda i: (0, 0, i, 0)),
            pl.BlockSpec((_BN, _D), lambda i: (i, 0)),
            pl.BlockSpec((_ED, _HD), lambda i: (0, 0)),
            pl.BlockSpec((_D, _HD), lambda i: (0, 0)),
            pl.BlockSpec((1, _HD), lambda i: (0, 0)),
            pl.BlockSpec((_HD, _D), lambda i: (0, 0)),
            pl.BlockSpec((1, _D), lambda i: (0, 0)),
            pl.BlockSpec((1, _D), lambda i: (0, 0)),
            pl.BlockSpec((1, _D), lambda i: (0, 0)),
        ],
        out_specs=pl.BlockSpec((_BN, _D), lambda i: (i, 0)),
        out_shape=jax.ShapeDtypeStruct((_N, _D), jnp.float32),
    )(zs, oh, x, we, wskip, bskip, wproj, bproj, gamma, beta)


_SR = 64
_SR1 = 16


def _fill_zeros(stg, rows):
    def _zb(i, carry):
        for t in range(_RW // 16):
            stg[i, pl.ds(t * 16, 16)] = jnp.zeros((16,), jnp.float32)
        return carry
    lax.fori_loop(0, rows, _zb, 0)


def _zero_slab(slab, stg, sid, rows):
    for j in range(_RPS // rows):
        pltpu.sync_copy(stg, slab.at[pl.ds(sid * _RPS + j * rows, rows)])


def _slab_to_hbm(slab, stg, out_hbm_rows, sid, rows):
    for j in range(_RPS // rows):
        r = sid * _RPS + j * rows
        pltpu.sync_copy(slab.at[pl.ds(r, rows)], stg)
        pltpu.sync_copy(stg, out_hbm_rows.at[pl.ds(r, rows)])


def _sc_p1_body(qg_hbm, k4_hbm, ea_hbm, src_hbm, dst_hbm,
                zs_hbm, p_hbm,
                slab, stg, dstbuf, srcbuf, qgbuf, k4buf, eabuf, resbuf,
                pbuf, sem_a, sem_b):
    cid = lax.axis_index("c")
    sid = lax.axis_index("s")
    wid = cid * _NS + sid
    lane = lax.iota(jnp.int32, 16)
    rot = [(lane + sh) & 15 for sh in (8, 4, 2, 1)]

    _fill_zeros(stg, _SR1)
    _zero_slab(slab, stg, sid, _SR1)
    plsc.subcore_barrier()

    def _base(cidx):
        return pl.multiple_of(wid * _EPW + cidx * _C1, 8)

    def _issue_idx(cidx, s):
        b = _base(cidx)
        return (
            pltpu.async_copy(dst_hbm.at[pl.ds(b, _C1)], dstbuf.at[s], sem_a),
            pltpu.async_copy(src_hbm.at[pl.ds(b, _C1)], srcbuf.at[s], sem_a),
            pltpu.async_copy(ea_hbm.at[pl.ds(b * _ED, _C1 * _ED)],
                             eabuf.at[s], sem_a),
        )

    def _wait_idx(s):
        pltpu.make_async_copy(dst_hbm.at[pl.ds(0, _C1)],
                              dstbuf.at[s], sem_a).wait()
        pltpu.make_async_copy(src_hbm.at[pl.ds(0, _C1)],
                              srcbuf.at[s], sem_a).wait()
        pltpu.make_async_copy(ea_hbm.at[pl.ds(0, _C1 * _ED)],
                              eabuf.at[s], sem_a).wait()

    def _issue_gather(s):
        pltpu.async_copy(qg_hbm.at[dstbuf.at[s]], qgbuf.at[s], sem_b)
        pltpu.async_copy(k4_hbm.at[srcbuf.at[s]], k4buf.at[s], sem_b)

    def _wait_gather(s):
        pltpu.make_async_copy(qg_hbm.at[dstbuf.at[s]],
                              qgbuf.at[s], sem_b).wait()
        pltpu.make_async_copy(k4_hbm.at[srcbuf.at[s]],
                              k4buf.at[s], sem_b).wait()

    for hnd in _issue_idx(0, 0):
        hnd.wait()
    _issue_gather(0)

    def _chunk(cidx, carry):
        s = lax.rem(cidx, 2)
        o = 1 - s
        not_last = cidx < _NCHUNK1 - 1

        @pl.when(not_last)
        def _():
            _issue_idx(cidx + 1, o)

        _wait_gather(s)

        def _edge(e, ecarry):
            ea = eabuf[s, pl.ds(e * _ED, _ED)]
            pv = []
            for h in range(_H):
                acc = ea * qgbuf[s, e, pl.ds(_HD + h * _ED, 16)]
                for r in range(8):
                    acc = acc + (qgbuf[s, e, pl.ds(h * _D + r * 16, 16)]
                                 * k4buf[s, e, pl.ds(h * _D + r * 16, 16)])
                for rv in rot:
                    acc = acc + jnp.take(acc, rv)
                p = jnp.exp(acc)
                resbuf[e, pl.ds(h * _ED, 16)] = p * ea
                resbuf[e, pl.ds(_H * _ED + h * _ED, 16)] = p
                pv.append(p)
            ps = pv[0]
            for h in range(1, _H):
                ps = jnp.where(lane == h, pv[h], ps)
            pbuf[pl.ds(e * 16, 16)] = ps
            return ecarry
        lax.fori_loop(0, _C1, _edge, 0)

        @pl.when(not_last)
        def _():
            _wait_idx(o)
            _issue_gather(o)

        pltpu.sync_copy(resbuf, slab.at[dstbuf.at[s]], add=True)
        pltpu.sync_copy(pbuf, p_hbm.at[pl.ds(_base(cidx) * 16, _C1 * 16)])
        return carry
    lax.fori_loop(0, _NCHUNK1, _chunk, 0)
    plsc.subcore_barrier()

    _slab_to_hbm(slab, stg, zs_hbm.at[cid], sid, _SR1)


_sc_p1 = functools.partial(
    pl.kernel,
    out_type=[
        jax.ShapeDtypeStruct((_NC, _NSLAB, _RW), jnp.float32),
        jax.ShapeDtypeStruct((_E * 16,), jnp.float32),
    ],
    mesh=plsc.VectorSubcoreMesh(core_axis_name="c", subcore_axis_name="s"),
    scratch_types=[
        pltpu.VMEM_SHARED((_NSLAB, _RW), jnp.float32),
        pltpu.VMEM((_SR1, _RW), jnp.float32),
        pltpu.VMEM((2, _C1), jnp.int32),
        pltpu.VMEM((2, _C1), jnp.int32),
        pltpu.VMEM((2, _C1, _QGW), jnp.float32),
        pltpu.VMEM((2, _C1, _HD), jnp.float32),
        pltpu.VMEM((2, _C1 * _ED), jnp.float32),
        pltpu.VMEM((_C1, _RW), jnp.float32),
        pltpu.VMEM((_C1 * 16,), jnp.float32),
        pltpu.SemaphoreType.DMA,
        pltpu.SemaphoreType.DMA,
    ],
)(_sc_p1_body)


def _sc_p2_body(v4_hbm, p_hbm, src_hbm, dst_hbm, oh_hbm,
                slab, stg, dstbuf, srcbuf, vidx, vbuf, pbuf, resbuf,
                sem_a, sem_b):
    cid = lax.axis_index("c")
    sid = lax.axis_index("s")
    wid = cid * _NS + sid

    for h in range(_H):
        _fill_zeros(stg, _SR)
        _zero_slab(slab, stg, sid, _SR)
        plsc.subcore_barrier()

        def _chunk(cidx, carry):
            base = pl.multiple_of(wid * _EPW + cidx * _C, 8)
            h1 = pltpu.async_copy(dst_hbm.at[pl.ds(base, _C)], dstbuf, sem_a)
            h2 = pltpu.async_copy(src_hbm.at[pl.ds(base, _C)], srcbuf, sem_a)
            h3 = pltpu.async_copy(p_hbm.at[pl.ds(base * 16, _C * 16)],
                                  pbuf, sem_a)
            h1.wait(); h2.wait(); h3.wait()
            for t in range(_C // 16):
                sv = srcbuf[pl.ds(t * 16, 16)]
                vidx[pl.ds(t * 16, 16)] = sv * _H + h
            pltpu.async_copy(v4_hbm.at[vidx], vbuf, sem_b).wait()

            hsel = jnp.full((16,), h, jnp.int32)

            def _edge(e, ecarry):
                prow = pbuf[pl.ds(e * 16, 16)]
                p = jnp.take(prow, hsel)
                for r in range(8):
                    resbuf[e, pl.ds(r * 16, 16)] = (
                        p * vbuf[e, pl.ds(r * 16, 16)])
                return ecarry
            lax.fori_loop(0, _C, _edge, 0)

            pltpu.sync_copy(resbuf, slab.at[dstbuf], add=True)
            return carry
        lax.fori_loop(0, _NCHUNK, _chunk, 0)
        plsc.subcore_barrier()

        _slab_to_hbm(slab, stg, oh_hbm.at[h, cid], sid, _SR)


_sc_p2 = functools.partial(
    pl.kernel,
    out_type=jax.ShapeDtypeStruct((_H, _NC, _NSLAB, _RW), jnp.float32),
    mesh=plsc.VectorSubcoreMesh(core_axis_name="c", subcore_axis_name="s"),
    scratch_types=[
        pltpu.VMEM_SHARED((_NSLAB, _RW), jnp.float32),
        pltpu.VMEM((_SR, _RW), jnp.float32),
        pltpu.VMEM((_C,), jnp.int32),
        pltpu.VMEM((_C,), jnp.int32),
        pltpu.VMEM((_C,), jnp.int32),
        pltpu.VMEM((_C, _D), jnp.float32),
        pltpu.VMEM((_C * 16,), jnp.float32),
        pltpu.VMEM((_C, _RW), jnp.float32),
        pltpu.SemaphoreType.DMA,
        pltpu.SemaphoreType.DMA,
    ],
)(_sc_p2_body)


def kernel(x, edge_index, edge_attr, Wq, bq, Wk, bk, Wv, bv, We, Wskip,
           bskip, Wproj, bproj, gamma, beta):
    src = edge_index[0].astype(jnp.int32)
    dst = edge_index[1].astype(jnp.int32)
    ea1d = edge_attr.reshape(-1)
    for l in range(3):
        we_l = We[l]
        wr = we_l.reshape(_ED, _H, _D)
        wet = jnp.zeros((_HD, _H * _ED), jnp.float32)
        for h in range(_H):
            wet = wet.at[h * _D:(h + 1) * _D,
                         h * _ED:(h + 1) * _ED].set(wr[:, h, :].T)
        qg, k4, v4 = _tca(x, Wq[l], bq[l].reshape(1, -1), Wk[l],
                          bk[l].reshape(1, -1), Wv[l], bv[l].reshape(1, -1),
                          wet)
        zs, p1d = _sc_p1(qg, k4, ea1d, src, dst)
        oh = _sc_p2(v4.reshape(_N * _H, _D), p1d, src, dst)
        x = _tcb(zs, oh, x, we_l, Wskip[l], bskip[l].reshape(1, -1),
                 Wproj[l], bproj[l].reshape(1, -1), gamma[l].reshape(1, -1),
                 beta[l].reshape(1, -1))
    return x
